# Initial kernel scaffold; baseline (speedup 1.0000x reference)
import jax
import jax.numpy as jnp
from jax.experimental import pallas as pl


def kernel(x, edge_index, edge_weight, batch, W1, b1, W2, b2, W3, b3):
    raise NotImplementedError("write your pallas kernel here")


# speedup vs baseline: 3.1884x; 3.1884x over previous
import jax
import jax.numpy as jnp
from jax import lax
from jax.experimental import pallas as pl
from jax.experimental.pallas import tpu as pltpu
from jax.experimental.pallas import tpu_sc as plsc

N_NODES = 10000
N_EDGES = 320000
FEAT = 128
CLS = 16
NUM_GRAPHS = 64
NS = 16
NC = 2
NW = NC * NS
RPT = 625
WIN = 632
CHUNK = 128
NCH = 79
EPAD = NW * NCH * CHUNK
NNCH = 3
NPADW = NW * NNCH * CHUNK
BATCH_PAD = 10112
POOL = 66 * FEAT
CNT_BASE = 65 * FEAT

_SC_PARAMS = pltpu.CompilerParams(needs_layout_passes=False)


def _aligned_win(s):
    return pl.multiple_of(s * RPT - lax.rem(s, 8), 8)


def _mm(x, w):
    bm = 1000

    def body(x_ref, w_ref, o_ref):
        o_ref[...] = jnp.dot(x_ref[...], w_ref[...],
                             preferred_element_type=jnp.float32)

    return pl.pallas_call(
        body,
        grid=(N_NODES // bm,),
        in_specs=[pl.BlockSpec((bm, FEAT), lambda i: (i, 0)),
                  pl.BlockSpec((FEAT, FEAT), lambda i: (0, 0))],
        out_specs=pl.BlockSpec((bm, FEAT), lambda i: (i, 0)),
        out_shape=jax.ShapeDtypeStruct((N_NODES, FEAT), jnp.float32),
    )(x, w)


def _fuse_relu_mm(y, acc, b2d, w):
    bm = 1000

    def body(y_ref, a_ref, b_ref, w_ref, o_ref):
        h = jnp.maximum(y_ref[...] + a_ref[0] + a_ref[1] + b_ref[...], 0.0)
        o_ref[...] = jnp.dot(h, w_ref[...], preferred_element_type=jnp.float32)

    return pl.pallas_call(
        body,
        grid=(N_NODES // bm,),
        in_specs=[pl.BlockSpec((bm, FEAT), lambda i: (i, 0)),
                  pl.BlockSpec((NC, bm, FEAT), lambda i: (0, i, 0)),
                  pl.BlockSpec((1, FEAT), lambda i: (0, 0)),
                  pl.BlockSpec((FEAT, FEAT), lambda i: (0, 0))],
        out_specs=pl.BlockSpec((bm, FEAT), lambda i: (i, 0)),
        out_shape=jax.ShapeDtypeStruct((N_NODES, FEAT), jnp.float32),
    )(y, acc, b2d, w)


def _fuse_relu(y, acc, b2d):
    bm = 1000

    def body(y_ref, a_ref, b_ref, o_ref):
        o_ref[...] = jnp.maximum(y_ref[...] + a_ref[0] + a_ref[1] + b_ref[...],
                                 0.0)

    return pl.pallas_call(
        body,
        grid=(N_NODES // bm,),
        in_specs=[pl.BlockSpec((bm, FEAT), lambda i: (i, 0)),
                  pl.BlockSpec((NC, bm, FEAT), lambda i: (0, i, 0)),
                  pl.BlockSpec((1, FEAT), lambda i: (0, 0))],
        out_specs=pl.BlockSpec((bm, FEAT), lambda i: (i, 0)),
        out_shape=jax.ShapeDtypeStruct((N_NODES, FEAT), jnp.float32),
    )(y, acc, b2d)


def _final_mm(pool4d, w3, b3):
    def body(p_ref, w_ref, b_ref, o_ref):
        s = jnp.sum(p_ref[...], axis=(0, 1))
        cnt = s[65, :NUM_GRAPHS]
        o_ref[...] = (jnp.dot(s[:NUM_GRAPHS], w_ref[...],
                              preferred_element_type=jnp.float32)
                      + cnt[:, None] * b_ref[...][None, 0, :])

    return pl.pallas_call(
        body,
        grid=(1,),
        in_specs=[pl.BlockSpec((NC, NS, 66, FEAT), lambda i: (0, 0, 0, 0)),
                  pl.BlockSpec((FEAT, CLS), lambda i: (0, 0)),
                  pl.BlockSpec((1, CLS), lambda i: (0, 0))],
        out_specs=pl.BlockSpec((NUM_GRAPHS, CLS), lambda i: (0, 0)),
        out_shape=jax.ShapeDtypeStruct((NUM_GRAPHS, CLS), jnp.float32),
    )(pool4d, w3, b3)


def _edge_acc(y, src2d, dst2d, wflat):
    mesh = plsc.VectorSubcoreMesh(core_axis_name="c", subcore_axis_name="s")

    def body(y_hbm, src_hbm, dst_hbm, w_hbm, out_hbm,
             acc_sh, src_v, dst_v, w_v, rows_v, gsem):
        c = lax.axis_index("c")
        s = lax.axis_index("s")
        wid = c * NS + s
        off = _aligned_win(s)
        pltpu.sync_copy(src_hbm.at[wid], src_v)
        pltpu.sync_copy(dst_hbm.at[wid], dst_v)
        pltpu.sync_copy(w_hbm.at[wid], w_v)

        zv = jnp.zeros((16,), jnp.float32)

        def zrow(e, cc):
            for f in range(FEAT // 16):
                rows_v[e, pl.ds(16 * f, 16)] = zv
            return cc

        lax.fori_loop(0, CHUNK, zrow, 0)
        for jo in range(5):
            sz = 128 if jo < 4 else WIN - 512
            base = pl.multiple_of(off + jo * 128, 8)
            pltpu.sync_copy(rows_v.at[pl.ds(0, sz)],
                            acc_sh.at[pl.ds(base, sz)])
        plsc.subcore_barrier()

        def chunk_body(j, carry):
            pltpu.async_copy(y_hbm.at[src_v.at[j]], rows_v, gsem).wait()

            def edge_body(e, c2):
                kv = jnp.full((16,), j * CHUNK + e, jnp.int32)
                wv = plsc.load_gather(w_v, [kv])
                for f in range(FEAT // 16):
                    sl = pl.ds(16 * f, 16)
                    rows_v[e, sl] = rows_v[e, sl] * wv
                return c2

            lax.fori_loop(0, CHUNK, edge_body, 0)
            pltpu.sync_copy(rows_v, acc_sh.at[dst_v.at[j]], add=True)
            return carry

        lax.fori_loop(0, NCH, chunk_body, 0)
        plsc.subcore_barrier()

        for jo in range(5):
            sz = 128 if jo < 4 else WIN - 512
            base = pl.multiple_of(off + jo * 128, 8)
            pltpu.sync_copy(acc_sh.at[pl.ds(base, sz)],
                            out_hbm.at[c, pl.ds(base, sz)])

    kern = pl.kernel(
        body,
        out_type=jax.ShapeDtypeStruct((NC, N_NODES, FEAT), jnp.float32),
        mesh=mesh,
        compiler_params=_SC_PARAMS,
        scratch_types=[
            pltpu.VMEM_SHARED((N_NODES, FEAT), jnp.float32),
            pltpu.VMEM((NCH, CHUNK), jnp.int32),
            pltpu.VMEM((NCH, CHUNK), jnp.int32),
            pltpu.VMEM((NCH * CHUNK,), jnp.float32),
            pltpu.VMEM((CHUNK, FEAT), jnp.float32),
            pltpu.SemaphoreType.DMA,
        ],
    )
    return kern(y, src2d, dst2d, wflat)


def _pool_acc(h2, src2d, dstflat, wflat, batch_ext, nid2d, nidflat):
    mesh = plsc.VectorSubcoreMesh(core_axis_name="c", subcore_axis_name="s")

    def body(h_hbm, src_hbm, dst_hbm, w_hbm, batch_hbm, nid_hbm, nidf_hbm,
             out_hbm, src_v, dst_v, w_v, batch_v, nid_v, nidf_v, rows_v,
             pool_v, gsem):
        c = lax.axis_index("c")
        s = lax.axis_index("s")
        wid = c * NS + s
        pltpu.sync_copy(src_hbm.at[wid], src_v)
        pltpu.sync_copy(dst_hbm.at[wid], dst_v)
        pltpu.sync_copy(w_hbm.at[wid], w_v)
        pltpu.sync_copy(batch_hbm, batch_v)
        pltpu.sync_copy(nid_hbm.at[wid], nid_v)
        pltpu.sync_copy(nidf_hbm.at[wid], nidf_v)

        iota16 = lax.iota(jnp.int32, 16)
        iotas = [iota16 + 16 * f for f in range(FEAT // 16)]
        zv = jnp.zeros((16,), jnp.float32)
        ones_v = jnp.ones((16,), jnp.float32)
        m0 = iota16 == 0

        def zbody(i, cc):
            plsc.store_scatter(pool_v, [i * 16 + iota16], zv)
            return cc

        lax.fori_loop(0, POOL // 16, zbody, 0)

        def chunk_body(j, carry):
            pltpu.async_copy(h_hbm.at[src_v.at[j]], rows_v, gsem).wait()

            def edge_body(e, c2):
                kv = jnp.full((16,), j * CHUNK + e, jnp.int32)
                wv = plsc.load_gather(w_v, [kv])
                dv = plsc.load_gather(dst_v, [kv])
                gv = plsc.load_gather(batch_v, [dv])
                gb = gv * FEAT
                for f in range(FEAT // 16):
                    v = rows_v[e, pl.ds(16 * f, 16)] * wv
                    plsc.addupdate_scatter(pool_v, [gb + iotas[f]], v)
                return c2

            lax.fori_loop(0, CHUNK, edge_body, 0)
            return carry

        lax.fori_loop(0, NCH, chunk_body, 0)

        def nchunk_body(j, carry):
            pltpu.async_copy(h_hbm.at[nid_v.at[j]], rows_v, gsem).wait()

            def node_body(e, c2):
                kv = jnp.full((16,), j * CHUNK + e, jnp.int32)
                nv = plsc.load_gather(nidf_v, [kv])
                gv = plsc.load_gather(batch_v, [nv])
                gb = gv * FEAT
                for f in range(FEAT // 16):
                    v = rows_v[e, pl.ds(16 * f, 16)]
                    plsc.addupdate_scatter(pool_v, [gb + iotas[f]], v)
                plsc.addupdate_scatter(pool_v, [gv + CNT_BASE], ones_v,
                                       mask=m0)
                return c2

            lax.fori_loop(0, CHUNK, node_body, 0)
            return carry

        lax.fori_loop(0, NNCH, nchunk_body, 0)
        pltpu.sync_copy(pool_v, out_hbm.at[c, s])

    kern = pl.kernel(
        body,
        out_type=jax.ShapeDtypeStruct((NC, NS, POOL), jnp.float32),
        mesh=mesh,
        compiler_params=_SC_PARAMS,
        scratch_types=[
            pltpu.VMEM((NCH, CHUNK), jnp.int32),
            pltpu.VMEM((NCH * CHUNK,), jnp.int32),
            pltpu.VMEM((NCH * CHUNK,), jnp.float32),
            pltpu.VMEM((BATCH_PAD,), jnp.int32),
            pltpu.VMEM((NNCH, CHUNK), jnp.int32),
            pltpu.VMEM((NNCH * CHUNK,), jnp.int32),
            pltpu.VMEM((CHUNK, FEAT), jnp.float32),
            pltpu.VMEM((POOL,), jnp.float32),
            pltpu.SemaphoreType.DMA,
        ],
    )
    return kern(h2, src2d, dstflat, wflat, batch_ext, nid2d, nidflat)


def kernel(x, edge_index, edge_weight, batch, W1, b1, W2, b2, W3, b3):
    src = edge_index[0].astype(jnp.int32)
    dst = edge_index[1].astype(jnp.int32)
    w = edge_weight.astype(jnp.float32)

    pad = EPAD - N_EDGES
    srcp = jnp.concatenate([src, jnp.zeros((pad,), jnp.int32)])
    dstp = jnp.concatenate([dst, jnp.zeros((pad,), jnp.int32)])
    wp = jnp.concatenate([w, jnp.zeros((pad,), jnp.float32)])
    src2d = srcp.reshape(NW, NCH, CHUNK)
    dst2d = dstp.reshape(NW, NCH, CHUNK)
    dstflat = dstp.reshape(NW, NCH * CHUNK)
    wflat = wp.reshape(NW, NCH * CHUNK)

    batch_ext = jnp.concatenate([
        batch.astype(jnp.int32),
        jnp.full((BATCH_PAD - N_NODES,), NUM_GRAPHS, jnp.int32)])

    npad = NPADW - N_NODES
    nid_g = jnp.concatenate([jnp.arange(N_NODES, dtype=jnp.int32),
                             jnp.zeros((npad,), jnp.int32)])
    nid_b = jnp.concatenate([jnp.arange(N_NODES, dtype=jnp.int32),
                             jnp.full((npad,), N_NODES, jnp.int32)])
    nid2d = nid_g.reshape(NW, NNCH, CHUNK)
    nidflat = nid_b.reshape(NW, NNCH * CHUNK)

    b1_2d = b1.reshape(1, FEAT)
    b2_2d = b2.reshape(1, FEAT)
    b3_2d = b3.reshape(1, CLS)

    y1 = _mm(x, W1)
    a1 = _edge_acc(y1, src2d, dst2d, wflat)
    y2 = _fuse_relu_mm(y1, a1, b1_2d, W2)
    a2 = _edge_acc(y2, src2d, dst2d, wflat)
    h2 = _fuse_relu(y2, a2, b2_2d)
    pool = _pool_acc(h2, src2d, dstflat, wflat, batch_ext, nid2d, nidflat)
    pool4d = pool.reshape(NC, NS, 66, FEAT)
    return _final_mm(pool4d, W3, b3_2d)
